# baseline (speedup 1.0000x reference)
import functools

import jax
import jax.numpy as jnp
from jax import lax
from jax.experimental import pallas as pl
from jax.experimental.pallas import tpu as pltpu
from jax.experimental.pallas import tpu_sc as plsc

B = 128
V = 100000
L = 16
NC, NS = 2, 16
NW = NC * NS
RPW = 4
WC = 4096
NPAIR = 12
TAIL0 = 2 * NPAIR * WC
TAILW = 1664
MICRO0 = TAIL0 + TAILW
MICROW = V - MICRO0
NV_CHUNK = WC // L
NV_TAIL = TAILW // L
NACC = 8
NEG = -3.0e38
LN2 = 0.6931471805599453


def _pass_max(buf, r, nv, step, m):
    init = tuple(jnp.full((L,), NEG, jnp.float32) for _ in range(NACC))

    @plsc.parallel_loop(0, nv, step=step, carry=init)
    def accs(i, accs):
        accs = list(accs)
        for k in range(step):
            x = buf[r, pl.ds((i + k) * L, L)]
            accs[k % NACC] = jnp.maximum(accs[k % NACC], x)
        return tuple(accs)

    cm = accs[0]
    for a in accs[1:]:
        cm = jnp.maximum(cm, a)
    return jnp.maximum(m, cm)


def _pass_sumexp(buf, r, nv, step, mx):
    init = tuple(jnp.zeros((L,), jnp.float32) for _ in range(NACC))

    @plsc.parallel_loop(0, nv, step=step, carry=init)
    def accs(i, accs):
        accs = list(accs)
        for k in range(step):
            x = buf[r, pl.ds((i + k) * L, L)]
            accs[k % NACC] = accs[k % NACC] + jnp.exp(x - mx)
        return tuple(accs)

    cs = accs[0]
    for a in accs[1:]:
        cs = cs + a
    return cs


def _allreduce16(x, t, op):
    for k in (8, 4, 2, 1):
        t[pl.ds(0, L)] = x
        t[pl.ds(L, L)] = x
        x = op(x, t[pl.ds(k, L)])
    return x


def _log16(x):
    b = lax.bitcast_convert_type(x, jnp.int32)
    e = ((b >> 23) & 0xFF) - 127
    f = lax.bitcast_convert_type((b & 0x7FFFFF) | 0x3F800000, jnp.float32)
    u = f - 1.0
    y = e.astype(jnp.float32) * LN2 + u * (1.0 - 0.5 * u)
    for _ in range(3):
        y = y + x * jnp.exp(-y) - 1.0
    return y


def _sc_body(logits_hbm, value_hbm, ext_hbm, out_hbm,
             buf0, buf1, tbuf, val_v, vg_v, st_v, red_t, out_v, ext_v,
             val_sh, ext_sh, val_s, ext_s,
             sem0, sem1, semt):
    wid = lax.axis_index("s") * NC + lax.axis_index("c")
    blk = wid // 2
    roff = (wid % 2) * RPW
    row0 = blk * 8
    iota = lax.iota(jnp.int32, L)

    pltpu.sync_copy(value_hbm, val_v)
    pltpu.sync_copy(val_v, val_sh)
    pltpu.sync_copy(val_sh, val_s)
    pltpu.sync_copy(ext_hbm, ext_v)
    pltpu.sync_copy(ext_v, ext_sh)
    pltpu.sync_copy(ext_sh, ext_s)
    vrow = [val_s[row0 + roff + rr] for rr in range(RPW)]

    zero = jnp.zeros((L,), jnp.float32)
    for rr in range(RPW):
        st_v[pl.ds(rr * L, L)] = jnp.full((L,), NEG, jnp.float32)
        st_v[pl.ds((RPW + rr) * L, L)] = zero
        vg_v[pl.ds(rr * L, L)] = zero

    def issue(c0, buf, sem, w):
        pltpu.make_async_copy(
            logits_hbm.at[pl.ds(row0, 8), pl.ds(c0, w)], buf, sem).start()

    def wait(buf, sem):
        pltpu.make_async_copy(
            logits_hbm.at[pl.ds(row0, 8), pl.ds(0, buf.shape[1])],
            buf, sem).wait()

    def process(buf, c0, nv, step):
        for rr in range(RPW):
            r = roff + rr
            m = st_v[pl.ds(rr * L, L)]
            s = st_v[pl.ds((RPW + rr) * L, L)]
            new_m = _pass_max(buf, r, nv, step, m)
            s = s * jnp.exp(m - new_m) + _pass_sumexp(buf, r, nv, step, new_m)
            st_v[pl.ds(rr * L, L)] = new_m
            st_v[pl.ds((RPW + rr) * L, L)] = s
            v = vrow[rr]
            rel = v - c0

            @pl.when((rel >= 0) & (rel < nv * L))
            def _pick():
                xv = buf[r, pl.ds((rel // L) * L, L)]
                vg_v[pl.ds(rr * L, L)] = jnp.where(iota == rel % L, xv, 0.0)

    issue(0, buf0, sem0, WC)
    issue(WC, buf1, sem1, WC)
    issue(TAIL0, tbuf, semt, TAILW)

    def pair(i, _):
        c0 = 2 * i * WC
        wait(buf0, sem0)
        process(buf0, c0, NV_CHUNK, 32)

        @pl.when(i < NPAIR - 1)
        def _i0():
            issue(c0 + 2 * WC, buf0, sem0, WC)

        wait(buf1, sem1)
        process(buf1, c0 + WC, NV_CHUNK, 32)

        @pl.when(i < NPAIR - 1)
        def _i1():
            issue(c0 + 3 * WC, buf1, sem1, WC)

        return _

    lax.fori_loop(0, NPAIR, pair, None)
    wait(tbuf, semt)
    process(tbuf, TAIL0, NV_TAIL, 52)

    gms, sts, vls = [], [], []
    for rr in range(RPW):
        row = row0 + roff + rr
        m = st_v[pl.ds(rr * L, L)]
        s = st_v[pl.ds((RPW + rr) * L, L)]
        gm = _allreduce16(m, red_t, jnp.maximum)
        st = _allreduce16(s * jnp.exp(m - gm), red_t, jnp.add)
        me = jnp.full((L,), ext_s[row])
        se = jnp.full((L,), ext_s[B + row])
        ve = jnp.full((L,), ext_s[2 * B + row])
        g2 = jnp.maximum(gm, me)
        st = st * jnp.exp(gm - g2) + se * jnp.exp(me - g2)
        vl = _allreduce16(vg_v[pl.ds(rr * L, L)], red_t, jnp.add)
        vl = jnp.where(vrow[rr] >= MICRO0, ve, vl)
        gms.append(g2)
        sts.append(st)
        vls.append(vl)
    gv, sv, vlog = gms[-1], sts[-1], vls[-1]
    for rr in range(RPW - 1):
        gv = jnp.where(iota == rr, gms[rr], gv)
        sv = jnp.where(iota == rr, sts[rr], sv)
        vlog = jnp.where(iota == rr, vls[rr], vlog)
    out_v[...] = vlog - gv - _log16(sv)
    pltpu.sync_copy(out_v, out_hbm.at[wid])


@jax.jit
def kernel(logits, value):
    mesh = plsc.VectorSubcoreMesh(
        core_axis_name="c", subcore_axis_name="s",
        num_cores=NC, num_subcores=NS)
    run = functools.partial(
        pl.kernel,
        out_type=jax.ShapeDtypeStruct((NW, L), jnp.float32),
        mesh=mesh,
        scratch_types=[
            pltpu.VMEM((8, WC), jnp.float32),
            pltpu.VMEM((8, WC), jnp.float32),
            pltpu.VMEM((8, TAILW), jnp.float32),
            pltpu.VMEM((B,), jnp.int32),
            pltpu.VMEM((RPW * L,), jnp.float32),
            pltpu.VMEM((2 * RPW * L,), jnp.float32),
            pltpu.VMEM((2 * L,), jnp.float32),
            pltpu.VMEM((L,), jnp.float32),
            pltpu.VMEM((3 * B,), jnp.float32),
            pltpu.VMEM_SHARED((B,), jnp.int32),
            pltpu.VMEM_SHARED((3 * B,), jnp.float32),
            pltpu.SMEM((B,), jnp.int32),
            pltpu.SMEM((3 * B,), jnp.float32),
            pltpu.SemaphoreType.DMA,
            pltpu.SemaphoreType.DMA,
            pltpu.SemaphoreType.DMA,
        ],
    )(_sc_body)
    vi = value.astype(jnp.int32)
    tail = lax.slice(logits, (0, MICRO0), (B, V))
    me = jnp.max(tail, axis=1)
    se = jnp.sum(jnp.exp(tail - me[:, None]), axis=1)
    vt = jnp.take_along_axis(
        tail, jnp.clip(vi[:, None] - MICRO0, 0, MICROW - 1), axis=1)[:, 0]
    ext = jnp.concatenate([me, se, vt])
    out2d = run(logits, vi, ext)
    return out2d[:, :RPW].reshape(B)


# speedup vs baseline: 1.5302x; 1.5302x over previous
import functools

import jax
import jax.numpy as jnp
from jax import lax
from jax.experimental import pallas as pl
from jax.experimental.pallas import tpu as pltpu
from jax.experimental.pallas import tpu_sc as plsc

B = 128
V = 100000
L = 16
NG = B // L
NC, NS = 2, 16
NW = NC * NS
RPW = 4
VR = 344
NCH = 9
TAILR = 32
NEG = -3.0e38
LN2 = 0.6931471805599453


def _allreduce16(x, t, op):
    for k in (8, 4, 2, 1):
        t[pl.ds(0, L)] = x
        t[pl.ds(L, L)] = x
        x = op(x, t[pl.ds(k, L)])
    return x


def _log16(x):
    b = lax.bitcast_convert_type(x, jnp.int32)
    e = ((b >> 23) & 0xFF) - 127
    f = lax.bitcast_convert_type((b & 0x7FFFFF) | 0x3F800000, jnp.float32)
    u = f - 1.0
    y = e.astype(jnp.float32) * LN2 + u * (1.0 - 0.5 * u)
    for _ in range(3):
        y = y + x * jnp.exp(-y) - 1.0
    return y


def _reduce_body(lt_hbm, part_hbm, buf0, buf1, tbuf, st_v, red_unused,
                 sem0, sem1, semt):
    wid = lax.axis_index("s") * NC + lax.axis_index("c")
    v0 = wid * 3120 + 8 * jnp.minimum(wid, 20)
    vlen = 3120 + 8 * jnp.where(wid < 20, 1, 0)

    for j in range(NG):
        st_v[pl.ds(j * L, L)] = jnp.full((L,), NEG, jnp.float32)
        st_v[pl.ds((NG + j) * L, L)] = jnp.zeros((L,), jnp.float32)

    def issue(row, buf, sem, nrows):
        pltpu.make_async_copy(
            lt_hbm.at[pl.ds(row, nrows), :], buf, sem).start()

    def wait(buf, sem, nrows):
        pltpu.make_async_copy(
            lt_hbm.at[pl.ds(0, nrows), :], buf, sem).wait()

    def process(buf, lo, hi):
        m = [st_v[pl.ds(j * L, L)] for j in range(NG)]
        s = [st_v[pl.ds((NG + j) * L, L)] for j in range(NG)]

        @plsc.parallel_loop(lo, hi, step=64, carry=tuple(m))
        def new_m(i, acc):
            acc = list(acc)
            rb = i // NG
            for r in range(8):
                for g in range(NG):
                    acc[g] = jnp.maximum(acc[g], buf[rb + r, pl.ds(g * L, L)])
            return tuple(acc)

        s = [s[j] * jnp.exp(m[j] - new_m[j]) for j in range(NG)]

        @plsc.parallel_loop(lo, hi, step=64, carry=tuple(s))
        def new_s(i, acc):
            acc = list(acc)
            rb = i // NG
            for r in range(8):
                for g in range(NG):
                    x = buf[rb + r, pl.ds(g * L, L)]
                    acc[g] = acc[g] + jnp.exp(x - new_m[g])
            return tuple(acc)

        for j in range(NG):
            st_v[pl.ds(j * L, L)] = new_m[j]
            st_v[pl.ds((NG + j) * L, L)] = new_s[j]

    tstart = jnp.minimum(v0 + NCH * VR, V - TAILR)
    tlo = (v0 + NCH * VR - tstart) * NG
    thi = tlo + (vlen - NCH * VR) * NG

    issue(v0, buf0, sem0, VR)
    issue(v0 + VR, buf1, sem1, VR)
    issue(tstart, tbuf, semt, TAILR)

    def pair(i, _):
        wait(buf0, sem0, VR)
        process(buf0, 0, VR * NG)

        @pl.when(i < 4)
        def _i0():
            issue(v0 + (2 * i + 2) * VR, buf0, sem0, VR)

        wait(buf1, sem1, VR)
        process(buf1, 0, VR * NG)

        @pl.when(i < 3)
        def _i1():
            issue(v0 + (2 * i + 3) * VR, buf1, sem1, VR)

        return _

    lax.fori_loop(0, 4, pair, None)
    wait(buf0, sem0, VR)
    process(buf0, 0, VR * NG)
    wait(tbuf, semt, TAILR)
    process(tbuf, tlo, thi)

    pltpu.sync_copy(st_v, part_hbm.at[wid])


def _combine_body(lt_hbm, value_hbm, part_hbm, out_hbm,
                  part_v, val_v, gbuf, out_v, red_t, val_sh, val_s, semg):
    wid = lax.axis_index("s") * NC + lax.axis_index("c")
    g0 = wid // 4
    iota = lax.iota(jnp.int32, L)

    pltpu.sync_copy(value_hbm, val_v)
    pltpu.sync_copy(val_v, val_sh)
    pltpu.sync_copy(val_sh, val_s)
    pltpu.sync_copy(part_hbm, part_v)

    vrow = []
    for rr in range(RPW):
        v = val_s[RPW * wid + rr]
        a8 = (v // 8) * 8
        pltpu.make_async_copy(
            lt_hbm.at[pl.ds(a8, 8), :], gbuf.at[pl.ds(rr * 8, 8), :],
            semg).start()
        vrow.append((v, a8))

    mt = [part_v[t, pl.ds(g0 * L, L)] for t in range(NW)]
    m_tot = mt[0]
    for t in range(1, NW):
        m_tot = jnp.maximum(m_tot, mt[t])
    s_tot = jnp.zeros((L,), jnp.float32)
    for t in range(NW):
        st = part_v[t, pl.ds((NG + g0) * L, L)]
        s_tot = s_tot + st * jnp.exp(mt[t] - m_tot)

    for rr in range(RPW):
        pltpu.make_async_copy(
            lt_hbm.at[pl.ds(0, 8), :], gbuf.at[pl.ds(rr * 8, 8), :],
            semg).wait()

    gv = jnp.zeros((L,), jnp.float32)
    sv = jnp.ones((L,), jnp.float32)
    vlog = jnp.zeros((L,), jnp.float32)
    for rr in range(RPW):
        lane = (RPW * wid + rr) % L
        v, a8 = vrow[rr]
        xg = gbuf[rr * 8 + (v - a8), pl.ds(g0 * L, L)]
        vl = _allreduce16(jnp.where(iota == lane, xg, 0.0), red_t, jnp.add)
        mr = _allreduce16(jnp.where(iota == lane, m_tot, 0.0), red_t, jnp.add)
        sr = _allreduce16(jnp.where(iota == lane, s_tot, 0.0), red_t, jnp.add)
        gv = jnp.where(iota == rr, mr, gv)
        sv = jnp.where(iota == rr, sr, sv)
        vlog = jnp.where(iota == rr, vl, vlog)
    out_v[...] = vlog - gv - _log16(sv)
    pltpu.sync_copy(out_v, out_hbm.at[wid])


@jax.jit
def kernel(logits, value):
    mesh = plsc.VectorSubcoreMesh(
        core_axis_name="c", subcore_axis_name="s",
        num_cores=NC, num_subcores=NS)
    lt = logits.T
    reduce_run = functools.partial(
        pl.kernel,
        out_type=jax.ShapeDtypeStruct((NW, 2 * NG * L), jnp.float32),
        mesh=mesh,
        scratch_types=[
            pltpu.VMEM((VR, B), jnp.float32),
            pltpu.VMEM((VR, B), jnp.float32),
            pltpu.VMEM((TAILR, B), jnp.float32),
            pltpu.VMEM((2 * NG * L,), jnp.float32),
            pltpu.VMEM((2 * L,), jnp.float32),
            pltpu.SemaphoreType.DMA,
            pltpu.SemaphoreType.DMA,
            pltpu.SemaphoreType.DMA,
        ],
    )(_reduce_body)
    combine_run = functools.partial(
        pl.kernel,
        out_type=jax.ShapeDtypeStruct((NW, L), jnp.float32),
        mesh=mesh,
        scratch_types=[
            pltpu.VMEM((NW, 2 * NG * L), jnp.float32),
            pltpu.VMEM((B,), jnp.int32),
            pltpu.VMEM((RPW * 8, B), jnp.float32),
            pltpu.VMEM((L,), jnp.float32),
            pltpu.VMEM((2 * L,), jnp.float32),
            pltpu.VMEM_SHARED((B,), jnp.int32),
            pltpu.SMEM((B,), jnp.int32),
            pltpu.SemaphoreType.DMA,
        ],
    )(_combine_body)
    vi = value.astype(jnp.int32)
    part = reduce_run(lt)
    out2d = combine_run(lt, vi, part)
    return out2d[:, :RPW].reshape(B)


# speedup vs baseline: 1.6801x; 1.0980x over previous
import functools

import jax
import jax.numpy as jnp
from jax import lax
from jax.experimental import pallas as pl
from jax.experimental.pallas import tpu as pltpu
from jax.experimental.pallas import tpu_sc as plsc

B = 128
V = 100000
L = 16
NG = B // L
NC, NS = 2, 16
NW = NC * NS
RPW = 4
VR = 344
NCH = 9
TAILR = 32
NEG = -3.0e38
LN2 = 0.6931471805599453


def _allreduce16(x, t, op):
    for k in (8, 4, 2, 1):
        t[pl.ds(0, L)] = x
        t[pl.ds(L, L)] = x
        x = op(x, t[pl.ds(k, L)])
    return x


def _log16(x):
    b = lax.bitcast_convert_type(x, jnp.int32)
    e = ((b >> 23) & 0xFF) - 127
    f = lax.bitcast_convert_type((b & 0x7FFFFF) | 0x3F800000, jnp.float32)
    u = f - 1.0
    y = e.astype(jnp.float32) * LN2 + u * (1.0 - 0.5 * u)
    for _ in range(3):
        y = y + x * jnp.exp(-y) - 1.0
    return y


def _reduce_body(lt_hbm, part_hbm, buf0, buf1, tbuf, st_v, red_unused,
                 sem0, sem1, semt):
    wid = lax.axis_index("s") * NC + lax.axis_index("c")
    v0 = wid * 3120 + 8 * jnp.minimum(wid, 20)
    vlen = 3120 + 8 * jnp.where(wid < 20, 1, 0)

    for j in range(NG):
        st_v[pl.ds(j * L, L)] = jnp.full((L,), NEG, jnp.float32)
        st_v[pl.ds((NG + j) * L, L)] = jnp.zeros((L,), jnp.float32)

    def issue(row, buf, sem, nrows):
        pltpu.make_async_copy(
            lt_hbm.at[pl.ds(row, nrows), :], buf, sem).start()

    def wait(buf, sem, nrows):
        pltpu.make_async_copy(
            lt_hbm.at[pl.ds(0, nrows), :], buf, sem).wait()

    def process(buf, lo, hi):
        m = [st_v[pl.ds(j * L, L)] for j in range(NG)]
        s = [st_v[pl.ds((NG + j) * L, L)] for j in range(NG)]

        @plsc.parallel_loop(lo, hi, step=32, carry=tuple(m))
        def new_m(i, acc):
            acc = list(acc)
            rb = i // NG
            for g in range(NG):
                x = [buf[rb + r, pl.ds(g * L, L)] for r in range(4)]
                t = jnp.maximum(jnp.maximum(x[0], x[1]),
                                jnp.maximum(x[2], x[3]))
                acc[g] = jnp.maximum(acc[g], t)
            return tuple(acc)

        s = [s[j] * jnp.exp(m[j] - new_m[j]) for j in range(NG)]

        @plsc.parallel_loop(lo, hi, step=32, carry=tuple(s))
        def new_s(i, acc):
            acc = list(acc)
            rb = i // NG
            for g in range(NG):
                e = [jnp.exp(buf[rb + r, pl.ds(g * L, L)] - new_m[g])
                     for r in range(4)]
                acc[g] = acc[g] + ((e[0] + e[1]) + (e[2] + e[3]))
            return tuple(acc)

        for j in range(NG):
            st_v[pl.ds(j * L, L)] = new_m[j]
            st_v[pl.ds((NG + j) * L, L)] = new_s[j]

    tstart = jnp.minimum(v0 + NCH * VR, V - TAILR)
    tlo = (v0 + NCH * VR - tstart) * NG
    thi = tlo + (vlen - NCH * VR) * NG

    issue(v0, buf0, sem0, VR)
    issue(v0 + VR, buf1, sem1, VR)
    issue(tstart, tbuf, semt, TAILR)

    def pair(i, _):
        wait(buf0, sem0, VR)
        process(buf0, 0, VR * NG)

        @pl.when(i < 4)
        def _i0():
            issue(v0 + (2 * i + 2) * VR, buf0, sem0, VR)

        wait(buf1, sem1, VR)
        process(buf1, 0, VR * NG)

        @pl.when(i < 3)
        def _i1():
            issue(v0 + (2 * i + 3) * VR, buf1, sem1, VR)

        return _

    lax.fori_loop(0, 4, pair, None)
    wait(buf0, sem0, VR)
    process(buf0, 0, VR * NG)
    wait(tbuf, semt, TAILR)
    process(tbuf, tlo, thi)

    pltpu.sync_copy(st_v, part_hbm.at[wid])


def _combine_body(lt_hbm, value_hbm, part_hbm, out_hbm,
                  part_v, val_v, gbuf, out_v, red_t, val_sh, val_s, semg):
    wid = lax.axis_index("s") * NC + lax.axis_index("c")
    g0 = wid // 4
    iota = lax.iota(jnp.int32, L)

    pltpu.sync_copy(value_hbm, val_v)
    pltpu.sync_copy(val_v, val_sh)
    pltpu.sync_copy(val_sh, val_s)
    pltpu.sync_copy(part_hbm, part_v)

    vrow = []
    for rr in range(RPW):
        v = val_s[RPW * wid + rr]
        a8 = (v // 8) * 8
        pltpu.make_async_copy(
            lt_hbm.at[pl.ds(a8, 8), :], gbuf.at[pl.ds(rr * 8, 8), :],
            semg).start()
        vrow.append((v, a8))

    mt = [part_v[t, pl.ds(g0 * L, L)] for t in range(NW)]
    m_tot = mt[0]
    for t in range(1, NW):
        m_tot = jnp.maximum(m_tot, mt[t])
    s_tot = jnp.zeros((L,), jnp.float32)
    for t in range(NW):
        st = part_v[t, pl.ds((NG + g0) * L, L)]
        s_tot = s_tot + st * jnp.exp(mt[t] - m_tot)

    for rr in range(RPW):
        pltpu.make_async_copy(
            lt_hbm.at[pl.ds(0, 8), :], gbuf.at[pl.ds(rr * 8, 8), :],
            semg).wait()

    gv = jnp.zeros((L,), jnp.float32)
    sv = jnp.ones((L,), jnp.float32)
    vlog = jnp.zeros((L,), jnp.float32)
    for rr in range(RPW):
        lane = (RPW * wid + rr) % L
        v, a8 = vrow[rr]
        xg = gbuf[rr * 8 + (v - a8), pl.ds(g0 * L, L)]
        vl = _allreduce16(jnp.where(iota == lane, xg, 0.0), red_t, jnp.add)
        mr = _allreduce16(jnp.where(iota == lane, m_tot, 0.0), red_t, jnp.add)
        sr = _allreduce16(jnp.where(iota == lane, s_tot, 0.0), red_t, jnp.add)
        gv = jnp.where(iota == rr, mr, gv)
        sv = jnp.where(iota == rr, sr, sv)
        vlog = jnp.where(iota == rr, vl, vlog)
    out_v[...] = vlog - gv - _log16(sv)
    pltpu.sync_copy(out_v, out_hbm.at[wid])


@jax.jit
def kernel(logits, value):
    mesh = plsc.VectorSubcoreMesh(
        core_axis_name="c", subcore_axis_name="s",
        num_cores=NC, num_subcores=NS)
    lt = logits.T
    reduce_run = functools.partial(
        pl.kernel,
        out_type=jax.ShapeDtypeStruct((NW, 2 * NG * L), jnp.float32),
        mesh=mesh,
        scratch_types=[
            pltpu.VMEM((VR, B), jnp.float32),
            pltpu.VMEM((VR, B), jnp.float32),
            pltpu.VMEM((TAILR, B), jnp.float32),
            pltpu.VMEM((2 * NG * L,), jnp.float32),
            pltpu.VMEM((2 * L,), jnp.float32),
            pltpu.SemaphoreType.DMA,
            pltpu.SemaphoreType.DMA,
            pltpu.SemaphoreType.DMA,
        ],
    )(_reduce_body)
    combine_run = functools.partial(
        pl.kernel,
        out_type=jax.ShapeDtypeStruct((NW, L), jnp.float32),
        mesh=mesh,
        scratch_types=[
            pltpu.VMEM((NW, 2 * NG * L), jnp.float32),
            pltpu.VMEM((B,), jnp.int32),
            pltpu.VMEM((RPW * 8, B), jnp.float32),
            pltpu.VMEM((L,), jnp.float32),
            pltpu.VMEM((2 * L,), jnp.float32),
            pltpu.VMEM_SHARED((B,), jnp.int32),
            pltpu.SMEM((B,), jnp.int32),
            pltpu.SemaphoreType.DMA,
        ],
    )(_combine_body)
    vi = value.astype(jnp.int32)
    part = reduce_run(lt)
    out2d = combine_run(lt, vi, part)
    return out2d[:, :RPW].reshape(B)
